# Initial kernel scaffold; baseline (speedup 1.0000x reference)
import jax
import jax.numpy as jnp
from jax.experimental import pallas as pl


def kernel(x, edge_index, W1, b1, W2, b2):
    raise NotImplementedError("write your pallas kernel here")


# speedup vs baseline: 3.0765x; 3.0765x over previous
import jax
import jax.numpy as jnp
from jax import lax
from jax.experimental import pallas as pl

N = 10000
D_HID = 128


def _kb_body(deg_ref, x_ref, w1_ref, y1_ref):
    dis = lax.rsqrt(deg_ref[...])
    xw = jnp.dot(x_ref[...], w1_ref[...], preferred_element_type=jnp.float32)
    y1_ref[...] = xw * dis


def _kd_body(deg_ref, s1_ref, y1_ref, w2_ref, b1_ref, y2_ref):
    dis = lax.rsqrt(deg_ref[...])
    h = jnp.maximum(dis * (s1_ref[...] + y1_ref[...]) + b1_ref[...], 0.0)
    y2_ref[...] = dis * jnp.dot(h, w2_ref[...], preferred_element_type=jnp.float32)


def _kf_body(deg_ref, s2_ref, y2_ref, b2_ref, out_ref):
    dis = lax.rsqrt(deg_ref[...])
    out_ref[...] = dis * (s2_ref[...] + y2_ref[...]) + b2_ref[...]


_kb = pl.pallas_call(_kb_body, out_shape=jax.ShapeDtypeStruct((N, D_HID), jnp.float32))
_kd = pl.pallas_call(_kd_body, out_shape=jax.ShapeDtypeStruct((N, 8), jnp.float32))
_kf = pl.pallas_call(_kf_body, out_shape=jax.ShapeDtypeStruct((N, 8), jnp.float32))


def kernel(x, edge_index, W1, b1, W2, b2):
    row = edge_index[0].astype(jnp.int32)
    col = edge_index[1].astype(jnp.int32)
    deg = jax.ops.segment_sum(jnp.ones(col.shape[0], jnp.float32), col, num_segments=N)
    degc = (deg + 1.0).reshape(N, 1)
    y1 = _kb(degc, x, W1)
    s1 = jax.ops.segment_sum(y1[row], col, num_segments=N)
    W2p = jnp.pad(W2, ((0, 0), (0, 6)))
    b2p = jnp.pad(b2, (0, 6)).reshape(1, 8)
    y2 = _kd(degc, s1, y1, W2p, b1.reshape(1, D_HID))
    s2 = jax.ops.segment_sum(y2[row], col, num_segments=N)
    outp = _kf(degc, s2, y2, b2p)
    return outp[:, :2]


# speedup vs baseline: 8.3457x; 2.7127x over previous
import functools

import jax
import jax.numpy as jnp
from jax import lax
from jax.experimental import pallas as pl
from jax.experimental.pallas import tpu as pltpu
from jax.experimental.pallas import tpu_sc as plsc

N = 10000
E = 320000
D_IN = 128
D_HID = 128
D_OUT = 2

NC, NS = 2, 16
NW = NC * NS
NRP = 10008
NAP = 10016
CB = 2000
VPB = CB // 16
EPW = E // NW

_mesh = plsc.VectorSubcoreMesh(core_axis_name="c", subcore_axis_name="s")
_scparams = pltpu.CompilerParams(needs_layout_passes=False)


def _make_agg(F, split_edges, gather):
    TBL = NRP * F
    ACC = NAP * F
    nb = (EPW if split_edges else E) // CB
    scratch = [pltpu.VMEM((CB,), jnp.int32)]
    if gather:
        scratch += [pltpu.VMEM((CB,), jnp.int32), pltpu.VMEM((TBL,), jnp.float32)]
    scratch += [pltpu.VMEM((ACC,), jnp.float32)]

    @functools.partial(
        pl.kernel,
        out_type=jax.ShapeDtypeStruct((NW, ACC), jnp.float32),
        mesh=_mesh,
        compiler_params=_scparams,
        scratch_types=scratch,
    )
    def agg(*refs):
        if gather:
            y_hbm, row_hbm, col_hbm, out_hbm, cidx, ridx, ystage, acc = refs
        else:
            col_hbm, out_hbm, cidx, acc = refs
        cid = lax.axis_index("c")
        sid = lax.axis_index("s")
        wid = cid * NS + sid

        def zero(i, _):
            acc[pl.ds(i * 16, 16)] = jnp.zeros((16,), jnp.float32)
            return 0
        lax.fori_loop(0, ACC // 16, zero, 0)

        if gather:
            pltpu.sync_copy(y_hbm.at[0] if split_edges else y_hbm.at[wid], ystage)

        base0 = wid * EPW if split_edges else 0
        ones = jnp.ones((16,), jnp.float32)

        def batch(b, _):
            base = base0 + b * CB
            pltpu.sync_copy(col_hbm.at[pl.ds(base, CB)], cidx)
            if gather:
                pltpu.sync_copy(row_hbm.at[pl.ds(base, CB)], ridx)

            def vec(v, _):
                cvec = cidx[pl.ds(v * 16, 16)] * F
                if gather:
                    rvec = ridx[pl.ds(v * 16, 16)] * F
                    for f in range(F):
                        vals = plsc.load_gather(ystage, [rvec + f])
                        plsc.addupdate_scatter(acc, [cvec + f], vals)
                else:
                    plsc.addupdate_scatter(acc, [cvec], ones)
                return 0
            lax.fori_loop(0, VPB, vec, 0)
            return 0
        lax.fori_loop(0, nb, batch, 0)
        pltpu.sync_copy(acc, out_hbm.at[wid])

    return agg


_deg_kernel = _make_agg(1, split_edges=True, gather=False)
_agg_l1 = _make_agg(4, split_edges=False, gather=True)
_agg_l2 = _make_agg(2, split_edges=True, gather=True)


def _dis_from(degp_ref):
    deg = jnp.sum(degp_ref[...], axis=1, keepdims=True)[:N] + 1.0
    return lax.rsqrt(deg)


def _kb_body(degp_ref, x_ref, w1_ref, y1_ref):
    xw = jnp.dot(x_ref[...], w1_ref[...], preferred_element_type=jnp.float32)
    y1_ref[...] = xw * _dis_from(degp_ref)


def _kd_body(degp_ref, s1_ref, y1_ref, w2_ref, b1_ref, y2_ref):
    dis = _dis_from(degp_ref)
    h = jnp.maximum(dis * (s1_ref[...] + y1_ref[...]) + b1_ref[...], 0.0)
    y2_ref[...] = dis * jnp.dot(h, w2_ref[...], preferred_element_type=jnp.float32)


def _kf_body(degp_ref, s2_ref, y2_ref, sel_ref, b2_ref, out_ref):
    dis = _dis_from(degp_ref)
    s2 = jnp.dot(s2_ref[...], sel_ref[...], preferred_element_type=jnp.float32)
    out_ref[...] = dis * (s2 + y2_ref[...]) + b2_ref[...]


_kb = pl.pallas_call(_kb_body, out_shape=jax.ShapeDtypeStruct((N, D_HID), jnp.float32))
_kd = pl.pallas_call(_kd_body, out_shape=jax.ShapeDtypeStruct((N, D_OUT), jnp.float32))
_kf = pl.pallas_call(_kf_body, out_shape=jax.ShapeDtypeStruct((N, D_OUT), jnp.float32))


def kernel(x, edge_index, W1, b1, W2, b2):
    row = edge_index[0].astype(jnp.int32)
    col = edge_index[1].astype(jnp.int32)

    degp = _deg_kernel(col)
    degp_t = degp.T

    y1 = _kb(degp_t, x, W1)
    y1ch = (jnp.pad(y1, ((0, NRP - N), (0, 0)))
            .reshape(NRP, NW, 4).transpose(1, 0, 2).reshape(NW, NRP * 4))
    s1p = _agg_l1(y1ch, row, col)
    s1 = (s1p.reshape(NW, NAP, 4)[:, :N, :]
          .transpose(1, 0, 2).reshape(N, D_HID))

    y2 = _kd(degp_t, s1, y1, W2, b1.reshape(1, D_HID))
    y2ch = jnp.pad(y2, ((0, NRP - N), (0, 0))).reshape(1, NRP * D_OUT)
    s2p = _agg_l2(y2ch, row, col)
    s2in = (s2p.reshape(NW, NAP, D_OUT)[:, :N, :]
            .transpose(1, 0, 2).reshape(N, NW * D_OUT))

    sel = jnp.tile(jnp.eye(D_OUT, dtype=jnp.float32), (NW, 1))
    return _kf(degp_t, s2in, y2, sel, b2.reshape(1, D_OUT))
